# baseline (speedup 1.0000x reference)
import functools

import jax
import jax.numpy as jnp
from jax import lax
from jax.experimental import pallas as pl
from jax.experimental.pallas import tpu as pltpu
from jax.experimental.pallas import tpu_sc as plsc

_N = 6144
_TILE = 256
_NT = _N // _TILE
_CUT2 = 25.0

_NC = 2
_NS = 16
_NW = _NC * _NS
_CHUNK = _N // _NW
_GROUPS = _CHUNK // 16
_NB = _N // 16


def _deg_body(gid_hbm, pos_hbm, out_hbm, gid_v, pos_v, deg_v):
    wid = lax.axis_index("s") * _NC + lax.axis_index("c")
    base = wid * _CHUNK
    pltpu.sync_copy(gid_hbm, gid_v)
    pltpu.sync_copy(pos_hbm, pos_v)

    def search_block(target, strict_upper):
        def body(_, state):
            lo_b, hi_b = state
            mid = (lo_b + hi_b) // 2
            hd = gid_v[pl.ds(mid * 16, 16)][0]
            go = (hd <= target) if strict_upper else (hd < target)
            return jnp.where(go, mid + 1, lo_b), jnp.where(go, hi_b, mid)
        lo_b, _ = lax.fori_loop(0, 9, body, (0, _NB))
        return jnp.maximum(lo_b - 1, 0)

    lane = lax.iota(jnp.int32, 16)

    def group_body(k, _):
        gbase = base + k * 16
        gidi = gid_v[pl.ds(gbase, 16)]
        pxi = pos_v[pl.ds(gbase, 16)]
        pyi = pos_v[pl.ds(_N + gbase, 16)]
        pzi = pos_v[pl.ds(2 * _N + gbase, 16)]
        sqi = pxi * pxi + pyi * pyi + pzi * pzi
        lane_idx = gbase + lane

        blk_lo = search_block(gidi[0], False)
        blk_hi = search_block(gidi[15], True)

        def jb_body(jb, deg):
            j0 = jb * 16
            gj = gid_v[pl.ds(j0, 16)]
            xj = pos_v[pl.ds(j0, 16)]
            yj = pos_v[pl.ds(_N + j0, 16)]
            zj = pos_v[pl.ds(2 * _N + j0, 16)]
            for l in range(16):
                xs, ys, zs = xj[l], yj[l], zj[l]
                sqj = xs * xs + ys * ys + zs * zs
                d2 = sqi + sqj - 2.0 * (pxi * xs + pyi * ys + pzi * zs)
                ok = (gidi == gj[l]) & (d2 < _CUT2) & (lane_idx != j0 + l)
                deg = deg + jnp.where(ok, 1.0, 0.0)
            return deg

        deg = lax.fori_loop(blk_lo, blk_hi + 1, jb_body,
                            jnp.zeros((16,), jnp.float32))
        deg_v[pl.ds(k * 16, 16)] = deg
        return 0

    lax.fori_loop(0, _GROUPS, group_body, 0)
    pltpu.sync_copy(deg_v, out_hbm.at[pl.ds(base, _CHUNK)])


_deg_kernel = functools.partial(
    pl.kernel,
    out_type=jax.ShapeDtypeStruct((_N,), jnp.float32),
    mesh=plsc.VectorSubcoreMesh(core_axis_name="c", subcore_axis_name="s",
                                num_cores=_NC, num_subcores=_NS),
    scratch_types=[
        pltpu.VMEM((_N,), jnp.int32),
        pltpu.VMEM((3 * _N,), jnp.float32),
        pltpu.VMEM((_CHUNK,), jnp.float32),
    ],
)(_deg_body)


def _mlp_dot_kernel(atoms_ref, deg_ref, emb_ref, wf1_ref, bf1_ref,
                    wf2_ref, bf2_ref, out_ref):
    embt = emb_ref[...]
    h = jax.lax.dot_general(embt, wf1_ref[...], (((1,), (1,)), ((), ())),
                            preferred_element_type=jnp.float32) + bf1_ref[...]
    hs = h * jax.nn.sigmoid(h)
    w2s = jnp.sum(wf2_ref[...], axis=0, keepdims=True)
    gtab = jnp.sum(hs * w2s, axis=1, keepdims=True)
    gtab = gtab + jnp.sum(bf2_ref[...])

    lane_iota = jax.lax.broadcasted_iota(jnp.int32, (1, 128), 1)

    def outer(t, acc):
        r0 = t * _TILE
        atoms_t = atoms_ref[pl.ds(r0, _TILE), :]
        onehot = (atoms_t == lane_iota).astype(jnp.float32)
        deg_t = deg_ref[pl.ds(r0, _TILE), :]
        return acc + jnp.sum(onehot * deg_t, axis=0, keepdims=True)

    acc = jax.lax.fori_loop(0, _NT, outer, jnp.zeros((1, 128), jnp.float32))
    tot = jax.lax.dot_general(acc, gtab, (((1,), (0,)), ((), ())),
                              preferred_element_type=jnp.float32)
    out_ref[0] = tot[0, 0]


def kernel(atoms, atom_positions, graph_indexes, emb, Wm1, bm1, Wm2, bm2,
           Wrbf, brbf, WU, WV, Wu1, bu1, Wu2, bu2, Wf1, bf1, Wf2, bf2):
    gid = graph_indexes.astype(jnp.int32)
    pos = atom_positions.astype(jnp.float32)
    pos_packed = pos.T.reshape(3 * _N)
    deg = _deg_kernel(gid, pos_packed)

    atoms2 = atoms.astype(jnp.int32).reshape(_N, 1)
    emb_p = jnp.zeros((128, 128), jnp.float32).at[:emb.shape[0]].set(emb)
    out = pl.pallas_call(
        _mlp_dot_kernel,
        out_shape=jax.ShapeDtypeStruct((1,), jnp.float32),
        in_specs=[pl.BlockSpec(memory_space=pltpu.VMEM)] * 7,
        out_specs=pl.BlockSpec(memory_space=pltpu.SMEM),
    )(atoms2, deg.reshape(_N, 1), emb_p, Wf1, bf1.reshape(1, -1),
      Wf2, bf2.reshape(1, -1))
    return out[0]


# speedup vs baseline: 1.2816x; 1.2816x over previous
import functools

import jax
import jax.numpy as jnp
from jax import lax
from jax.experimental import pallas as pl
from jax.experimental.pallas import tpu as pltpu
from jax.experimental.pallas import tpu_sc as plsc

_N = 6144
_TILE = 256
_NT = _N // _TILE
_CUT2 = 25.0

_NC = 2
_NS = 16
_NW = _NC * _NS
_CHUNK = _N // _NW
_GROUPS = _CHUNK // 16
_NB = _N // 16


def _deg_body(gid_hbm, pos_hbm, out_hbm, gid_v, pos_v, deg_v, sem1, sem2):
    wid = lax.axis_index("s") * _NC + lax.axis_index("c")
    base = wid * _CHUNK
    c1 = pltpu.async_copy(gid_hbm, gid_v, sem1)
    c2 = pltpu.async_copy(pos_hbm, pos_v, sem2)
    c1.wait()
    c2.wait()

    def search_block(target, strict_upper):
        def body(_, state):
            lo_b, hi_b = state
            mid = (lo_b + hi_b) // 2
            hd = gid_v[pl.ds(mid * 16, 16)][0]
            go = (hd <= target) if strict_upper else (hd < target)
            return jnp.where(go, mid + 1, lo_b), jnp.where(go, hi_b, mid)
        lo_b, _ = lax.fori_loop(0, 9, body, (0, _NB))
        return jnp.maximum(lo_b - 1, 0)

    lane = lax.iota(jnp.int32, 16)
    perms = [lax.rem(lane + r, jnp.int32(16)) for r in range(16)]
    _dnums = lax.GatherDimensionNumbers(
        offset_dims=(), collapsed_slice_dims=(0,), start_index_map=(0,))

    def rot(v, p):
        return lax.gather(v, p[:, None], _dnums, (1,),
                          mode=lax.GatherScatterMode.PROMISE_IN_BOUNDS)

    def group_body(k, _):
        gbase = base + k * 16
        gidi = gid_v[pl.ds(gbase, 16)]
        pxi = pos_v[pl.ds(gbase, 16)]
        pyi = pos_v[pl.ds(_N + gbase, 16)]
        pzi = pos_v[pl.ds(2 * _N + gbase, 16)]
        sqi = pxi * pxi + pyi * pyi + pzi * pzi
        lane_idx = gbase + lane

        blk_lo = search_block(gidi[0], False)
        blk_hi = search_block(gidi[15], True)

        def jb_body(jb, deg):
            j0 = jb * 16
            gj = gid_v[pl.ds(j0, 16)]
            xj = pos_v[pl.ds(j0, 16)]
            yj = pos_v[pl.ds(_N + j0, 16)]
            zj = pos_v[pl.ds(2 * _N + j0, 16)]
            sqj = xj * xj + yj * yj + zj * zj
            for r in range(16):
                p = perms[r]
                gr = rot(gj, p)
                xr = rot(xj, p)
                yr = rot(yj, p)
                zr = rot(zj, p)
                sr = rot(sqj, p)
                d2 = sqi + sr - 2.0 * (pxi * xr + pyi * yr + pzi * zr)
                ok = (gidi == gr) & (d2 < _CUT2) & (lane_idx != j0 + p)
                deg = deg + jnp.where(ok, 1.0, 0.0)
            return deg

        deg = lax.fori_loop(blk_lo, blk_hi + 1, jb_body,
                            jnp.zeros((16,), jnp.float32))
        deg_v[pl.ds(k * 16, 16)] = deg
        return 0

    lax.fori_loop(0, _GROUPS, group_body, 0)
    pltpu.sync_copy(deg_v, out_hbm.at[pl.ds(base, _CHUNK)])


_deg_kernel = functools.partial(
    pl.kernel,
    out_type=jax.ShapeDtypeStruct((_N,), jnp.float32),
    mesh=plsc.VectorSubcoreMesh(core_axis_name="c", subcore_axis_name="s",
                                num_cores=_NC, num_subcores=_NS),
    scratch_types=[
        pltpu.VMEM((_N,), jnp.int32),
        pltpu.VMEM((3 * _N,), jnp.float32),
        pltpu.VMEM((_CHUNK,), jnp.float32),
        pltpu.SemaphoreType.DMA,
        pltpu.SemaphoreType.DMA,
    ],
)(_deg_body)


def _mlp_dot_kernel(atoms_ref, deg_ref, emb_ref, wf1_ref, bf1_ref,
                    wf2_ref, bf2_ref, out_ref):
    embt = emb_ref[...]
    h = jax.lax.dot_general(embt, wf1_ref[...], (((1,), (1,)), ((), ())),
                            preferred_element_type=jnp.float32) + bf1_ref[...]
    hs = h * jax.nn.sigmoid(h)
    w2s = jnp.sum(wf2_ref[...], axis=0, keepdims=True)
    gtab = jnp.sum(hs * w2s, axis=1, keepdims=True)
    gtab = gtab + jnp.sum(bf2_ref[...])

    col_iota = jax.lax.broadcasted_iota(jnp.int32, (128, 1), 0)

    def outer(t, acc):
        r0 = t * _TILE
        atoms_t = atoms_ref[:, pl.ds(r0, _TILE)]
        onehot_t = (col_iota == atoms_t).astype(jnp.float32)
        deg_t = deg_ref[:, pl.ds(r0, _TILE)]
        return acc + jax.lax.dot_general(
            onehot_t, deg_t, (((1,), (1,)), ((), ())),
            preferred_element_type=jnp.float32)

    acc = jax.lax.fori_loop(0, _NT, outer, jnp.zeros((128, 1), jnp.float32))
    out_ref[0] = jnp.sum(acc * gtab)


def kernel(atoms, atom_positions, graph_indexes, emb, Wm1, bm1, Wm2, bm2,
           Wrbf, brbf, WU, WV, Wu1, bu1, Wu2, bu2, Wf1, bf1, Wf2, bf2):
    gid = graph_indexes.astype(jnp.int32)
    pos = atom_positions.astype(jnp.float32)
    pos_packed = pos.T.reshape(3 * _N)
    deg = _deg_kernel(gid, pos_packed)

    atoms2 = atoms.astype(jnp.int32).reshape(1, _N)
    emb_p = jnp.zeros((128, 128), jnp.float32).at[:emb.shape[0]].set(emb)
    out = pl.pallas_call(
        _mlp_dot_kernel,
        out_shape=jax.ShapeDtypeStruct((1,), jnp.float32),
        in_specs=[pl.BlockSpec(memory_space=pltpu.VMEM)] * 7,
        out_specs=pl.BlockSpec(memory_space=pltpu.SMEM),
    )(atoms2, deg.reshape(1, _N), emb_p, Wf1, bf1.reshape(1, -1),
      Wf2, bf2.reshape(1, -1))
    return out[0]


# speedup vs baseline: 1.3471x; 1.0511x over previous
import functools

import jax
import jax.numpy as jnp
from jax import lax
from jax.experimental import pallas as pl
from jax.experimental.pallas import tpu as pltpu
from jax.experimental.pallas import tpu_sc as plsc

_N = 6144
_TILE = 256
_NT = _N // _TILE
_CUT2 = 25.0

_NC = 2
_NS = 16
_NW = _NC * _NS
_CHUNK = _N // _NW
_GROUPS = _CHUNK // 16
_NB = _N // 16


def _deg_body(gid_hbm, pos_hbm, out_hbm, gid_v, pos_v, deg_v, sem1, sem2):
    wid = lax.axis_index("s") * _NC + lax.axis_index("c")
    base = wid * _CHUNK
    c1 = pltpu.async_copy(gid_hbm, gid_v, sem1)
    c2 = pltpu.async_copy(pos_hbm, pos_v, sem2)
    c1.wait()
    c2.wait()

    def search_block(target, strict_upper):
        def body(_, state):
            lo_b, hi_b = state
            mid = (lo_b + hi_b) // 2
            hd = gid_v[pl.ds(mid * 16, 16)][0]
            go = (hd <= target) if strict_upper else (hd < target)
            return jnp.where(go, mid + 1, lo_b), jnp.where(go, hi_b, mid)
        lo_b, _ = lax.fori_loop(0, 9, body, (0, _NB))
        return jnp.maximum(lo_b - 1, 0)

    lane = lax.iota(jnp.int32, 16)
    _dnums = lax.GatherDimensionNumbers(
        offset_dims=(), collapsed_slice_dims=(0,), start_index_map=(0,))

    def rot(v, p):
        return lax.gather(v, p[:, None], _dnums, (1,),
                          mode=lax.GatherScatterMode.PROMISE_IN_BOUNDS)

    def group_body(k, _):
        gbase = base + k * 16
        gidi = gid_v[pl.ds(gbase, 16)]
        pxi = pos_v[pl.ds(gbase, 16)]
        pyi = pos_v[pl.ds(_N + gbase, 16)]
        pzi = pos_v[pl.ds(2 * _N + gbase, 16)]
        sqi = pxi * pxi + pyi * pyi + pzi * pzi
        lane_idx = gbase + lane

        blk_lo = search_block(gidi[0], False)
        blk_hi = search_block(gidi[15], True)

        def jb_body(jb, deg):
            j0 = jb * 16
            gj = gid_v[pl.ds(j0, 16)]
            xj = pos_v[pl.ds(j0, 16)]
            yj = pos_v[pl.ds(_N + j0, 16)]
            zj = pos_v[pl.ds(2 * _N + j0, 16)]
            sqj = xj * xj + yj * yj + zj * zj

            def r_body(r, deg_r):
                p = lax.rem(lane + r, jnp.int32(16))
                gr = rot(gj, p)
                xr = rot(xj, p)
                yr = rot(yj, p)
                zr = rot(zj, p)
                sr = rot(sqj, p)
                d2 = sqi + sr - 2.0 * (pxi * xr + pyi * yr + pzi * zr)
                ok = (gidi == gr) & (d2 < _CUT2) & (lane_idx != j0 + p)
                return deg_r + jnp.where(ok, 1.0, 0.0)

            return lax.fori_loop(0, 16, r_body, deg)

        deg = lax.fori_loop(blk_lo, blk_hi + 1, jb_body,
                            jnp.zeros((16,), jnp.float32))
        deg_v[pl.ds(k * 16, 16)] = deg
        return 0

    lax.fori_loop(0, _GROUPS, group_body, 0)
    pltpu.sync_copy(deg_v, out_hbm.at[pl.ds(base, _CHUNK)])


_deg_kernel = functools.partial(
    pl.kernel,
    out_type=jax.ShapeDtypeStruct((_N,), jnp.float32),
    mesh=plsc.VectorSubcoreMesh(core_axis_name="c", subcore_axis_name="s",
                                num_cores=_NC, num_subcores=_NS),
    scratch_types=[
        pltpu.VMEM((_N,), jnp.int32),
        pltpu.VMEM((3 * _N,), jnp.float32),
        pltpu.VMEM((_CHUNK,), jnp.float32),
        pltpu.SemaphoreType.DMA,
        pltpu.SemaphoreType.DMA,
    ],
)(_deg_body)


def _mlp_dot_kernel(atoms_ref, deg_ref, emb_ref, wf1_ref, bf1_ref,
                    wf2_ref, bf2_ref, out_ref):
    embt = emb_ref[...]
    h = jax.lax.dot_general(embt, wf1_ref[...], (((1,), (1,)), ((), ())),
                            preferred_element_type=jnp.float32) + bf1_ref[...]
    hs = h * jax.nn.sigmoid(h)
    w2s = jnp.sum(wf2_ref[...], axis=0, keepdims=True)
    gtab = jnp.sum(hs * w2s, axis=1, keepdims=True)
    gtab = gtab + jnp.sum(bf2_ref[...])

    col_iota = jax.lax.broadcasted_iota(jnp.int32, (128, 1), 0)
    onehot = (col_iota == atoms_ref[...]).astype(jnp.float32)
    acc = jax.lax.dot_general(onehot, deg_ref[...], (((1,), (1,)), ((), ())),
                              preferred_element_type=jnp.float32)
    out_ref[0] = jnp.sum(acc * gtab)


def kernel(atoms, atom_positions, graph_indexes, emb, Wm1, bm1, Wm2, bm2,
           Wrbf, brbf, WU, WV, Wu1, bu1, Wu2, bu2, Wf1, bf1, Wf2, bf2):
    gid = graph_indexes.astype(jnp.int32)
    pos = atom_positions.astype(jnp.float32)
    pos_packed = pos.T.reshape(3 * _N)
    deg = _deg_kernel(gid, pos_packed)

    atoms2 = atoms.astype(jnp.int32).reshape(1, _N)
    emb_p = jnp.zeros((128, 128), jnp.float32).at[:emb.shape[0]].set(emb)
    out = pl.pallas_call(
        _mlp_dot_kernel,
        out_shape=jax.ShapeDtypeStruct((1,), jnp.float32),
        in_specs=[pl.BlockSpec(memory_space=pltpu.VMEM)] * 7,
        out_specs=pl.BlockSpec(memory_space=pltpu.SMEM),
    )(atoms2, deg.reshape(1, _N), emb_p, Wf1, bf1.reshape(1, -1),
      Wf2, bf2.reshape(1, -1))
    return out[0]
